# baseline (speedup 1.0000x reference)
import jax
import jax.numpy as jnp
from jax import lax
from jax.experimental import pallas as pl
from jax.experimental.pallas import tpu as pltpu

N, M, D = 16384, 8, 512
BN = 1024
NB = N // BN
KCP = 8
CPN = N // KCP


def _body(x_hbm, o_hbm, vin, vout, sin, sout, scp):
    def cp(k):
        return pltpu.make_async_copy(
            x_hbm.at[pl.ds(k * CPN, CPN), pl.ds(3, M - 3), :],
            o_hbm.at[pl.ds(k * CPN, CPN), pl.ds(3, M - 3), :],
            scp.at[k],
        )

    for k in range(KCP):
        cp(k).start()

    def in_copy(r, slot, i):
        return pltpu.make_async_copy(
            x_hbm.at[pl.ds(i * BN, BN), r, :], vin.at[r, slot], sin.at[r, slot]
        )

    def out_copy(r, slot, i):
        return pltpu.make_async_copy(
            vout.at[r, slot], o_hbm.at[pl.ds(i * BN, BN), r, :], sout.at[r, slot]
        )

    for r in range(3):
        in_copy(r, 0, 0).start()

    for i in range(NB):
        slot = i % 2
        for r in range(3):
            if i + 1 < NB:
                in_copy(r, (i + 1) % 2, i + 1).start()
            in_copy(r, slot, i).wait()
            if i >= 2:
                out_copy(r, slot, i - 2).wait()
            vout[r, slot] = vin[r, slot] * float(r + 2)
            out_copy(r, slot, i).start()

    for i in (NB - 2, NB - 1):
        for r in range(3):
            out_copy(r, i % 2, i).wait()

    for k in range(KCP):
        cp(k).wait()


def kernel(x):
    return pl.pallas_call(
        _body,
        in_specs=[pl.BlockSpec(memory_space=pl.ANY)],
        out_specs=pl.BlockSpec(memory_space=pl.ANY),
        out_shape=jax.ShapeDtypeStruct(x.shape, x.dtype),
        scratch_shapes=[
            pltpu.VMEM((3, 2, BN, D), jnp.float32),
            pltpu.VMEM((3, 2, BN, D), jnp.float32),
            pltpu.SemaphoreType.DMA((3, 2)),
            pltpu.SemaphoreType.DMA((3, 2)),
            pltpu.SemaphoreType.DMA((KCP,)),
        ],
    )(x)


# speedup vs baseline: 23.7333x; 23.7333x over previous
import functools
import jax
import jax.numpy as jnp
from jax import lax
from jax.experimental import pallas as pl
from jax.experimental.pallas import tpu as pltpu
from jax.experimental.pallas import tpu_sc as plsc

N, M, D = 16384, 8, 512
NC, NS = 2, 16
NW = NC * NS
ROWS_W = N // NW
C = 8
NCH = ROWS_W // C
NBUF = 3
L = 16


def _sc_body(x_hbm, o_hbm, buf, sin, sout):
    cid = lax.axis_index("c")
    sid = lax.axis_index("s")
    base = (sid * NC + cid) * ROWS_W

    def in_cp(slot, g):
        return pltpu.make_async_copy(
            x_hbm.at[pl.ds(base + g * C, C)], buf.at[slot], sin.at[slot]
        )

    def out_cp(slot, g):
        return pltpu.make_async_copy(
            buf.at[slot], o_hbm.at[pl.ds(base + g * C, C)], sout.at[slot]
        )

    def compute(slot):
        def body(k, _):
            c = k // 3
            r = k - 3 * c
            s = (r + 2).astype(jnp.float32)
            for j in range(D // L):
                v = buf[slot, c, r, pl.ds(j * L, L)]
                buf[slot, c, r, pl.ds(j * L, L)] = v * s
            return 0

        lax.fori_loop(0, C * 3, body, 0)

    for b in range(NBUF):
        in_cp(b, b).start()
    for g in range(NCH):
        slot = g % NBUF
        in_cp(slot, g).wait()
        compute(slot)
        out_cp(slot, g).start()
        nxt = g + NBUF
        if nxt < NCH:
            out_cp(slot, g).wait()
            in_cp(slot, nxt).start()
    for g in range(NCH - NBUF, NCH):
        out_cp(g % NBUF, g).wait()


_sc_call = pl.kernel(
    _sc_body,
    out_type=jax.ShapeDtypeStruct((N, M, D), jnp.float32),
    mesh=plsc.VectorSubcoreMesh(core_axis_name="c", subcore_axis_name="s"),
    scratch_types=[
        pltpu.VMEM((NBUF, C, M, D), jnp.float32),
        pltpu.SemaphoreType.DMA((NBUF,)),
        pltpu.SemaphoreType.DMA((NBUF,)),
    ],
)


def kernel(x):
    return _sc_call(x)


# speedup vs baseline: 30.2958x; 1.2765x over previous
import jax
import jax.numpy as jnp
from jax import lax
from jax.experimental import pallas as pl
from jax.experimental.pallas import tpu as pltpu

N, M, D = 16384, 8, 512
BN = 256
NCH = N // BN
NBUF = 4
PF = 2


def _body(x_hbm, o_hbm, buf, sin, sout):
    def in_cp(g, slot):
        return pltpu.make_async_copy(
            x_hbm.at[pl.ds(g * BN, BN)], buf.at[slot], sin.at[slot]
        )

    def out_cp(g, slot):
        return pltpu.make_async_copy(
            buf.at[slot], o_hbm.at[pl.ds(g * BN, BN)], sout.at[slot]
        )

    for g in range(PF):
        in_cp(g, g % NBUF).start()

    def step(g, _):
        slot = lax.rem(g, NBUF)
        in_cp(g, slot).wait()
        for r in range(3):
            buf[slot, :, r, :] = buf[slot, :, r, :] * float(r + 2)
        out_cp(g, slot).start()
        p = g + PF

        @pl.when(p < NCH)
        def _():
            pslot = lax.rem(p, NBUF)

            @pl.when(p >= NBUF)
            def _():
                out_cp(p - NBUF, pslot).wait()

            in_cp(p, pslot).start()

        return 0

    lax.fori_loop(0, NCH, step, 0)

    for g in range(NCH - NBUF, NCH):
        out_cp(g, g % NBUF).wait()


def kernel(x):
    return pl.pallas_call(
        _body,
        in_specs=[pl.BlockSpec(memory_space=pl.ANY)],
        out_specs=pl.BlockSpec(memory_space=pl.ANY),
        out_shape=jax.ShapeDtypeStruct(x.shape, x.dtype),
        scratch_shapes=[
            pltpu.VMEM((NBUF, BN, M, D), jnp.float32),
            pltpu.SemaphoreType.DMA((NBUF,)),
            pltpu.SemaphoreType.DMA((NBUF,)),
        ],
    )(x)


# speedup vs baseline: 30.6257x; 1.0109x over previous
import jax
import jax.numpy as jnp
from jax import lax
from jax.experimental import pallas as pl

_BLOCK_N = 512


def _scale_body(x_ref, o_ref):
    xb = x_ref[...]
    i = lax.broadcasted_iota(jnp.int32, xb.shape, 1)
    scale = jnp.where(
        i == 0, 2.0, jnp.where(i == 1, 3.0, jnp.where(i == 2, 4.0, 1.0))
    )
    o_ref[...] = xb * scale


def kernel(x):
    n, m, d = x.shape
    grid = (n // _BLOCK_N,)
    return pl.pallas_call(
        _scale_body,
        grid=grid,
        in_specs=[pl.BlockSpec((_BLOCK_N, m, d), lambda i: (i, 0, 0))],
        out_specs=pl.BlockSpec((_BLOCK_N, m, d), lambda i: (i, 0, 0)),
        out_shape=jax.ShapeDtypeStruct(x.shape, x.dtype),
    )(x)
